# baseline (speedup 1.0000x reference)
import functools

import jax
import jax.numpy as jnp
from jax import lax
from jax.experimental import pallas as pl
from jax.experimental.pallas import tpu as pltpu
from jax.experimental.pallas import tpu_sc as plsc

D_MODEL = 64
CHUNK = 128
K = 1
NBW = CHUNK // 32


@functools.cache
def _make_lookup(n_idx: int, d: int):
    info = plsc.get_sparse_core_info()
    nw = info.num_cores * info.num_subcores
    assert n_idx % (nw * CHUNK) == 0
    n_chunks = n_idx // (nw * CHUNK)
    n_groups = n_chunks // K
    assert n_chunks % K == 0 and n_groups % 2 == 0 and n_groups >= 4
    nbr = -(n_chunks * NBW // -128)
    mesh = plsc.VectorSubcoreMesh(core_axis_name="c", subcore_axis_name="s")

    @functools.partial(
        pl.kernel,
        mesh=mesh,
        out_type=jax.ShapeDtypeStruct((n_idx, d), jnp.float32),
        scratch_types=[
            pltpu.VMEM((n_chunks, CHUNK), jnp.int32),
            pltpu.VMEM((nbr, 128), jnp.int32),
            pltpu.VMEM((2, K, CHUNK, 2 * d), jnp.float32),
            pltpu.VMEM((2, K, CHUNK, d), jnp.float32),
            pltpu.SemaphoreType.DMA,
            pltpu.SemaphoreType.DMA,
            pltpu.SemaphoreType.DMA,
            pltpu.SemaphoreType.DMA,
        ],
        compiler_params=pltpu.CompilerParams(use_tc_tiling_on_sc=True),
    )
    def lookup(idxp_hbm, bits_hbm, table_hbm, out_hbm,
               idxp_v, bits_v, packed_v, ga, gb, oa, ob):
        wid = lax.axis_index("s") * info.num_cores + lax.axis_index("c")
        gsem = (ga, gb)
        osem = (oa, ob)
        pltpu.sync_copy(idxp_hbm.at[wid], idxp_v)
        pltpu.sync_copy(bits_hbm.at[wid], bits_v)

        def gather(t, p, k):
            return pltpu.make_async_copy(
                table_hbm.at[idxp_v.at[t * K + k]], packed_v.at[p, k], gsem[p]
            )

        def write(t, p, k):
            base = (wid * n_chunks + t * K + k) * CHUNK
            return pltpu.make_async_copy(
                compact_v.at[p, k], out_hbm.at[pl.ds(base, CHUNK)], osem[p]
            )

        def extract(t, p, k):
            c = t * K + k

            def body(j, carry):
                w_ix = c * NBW + lax.shift_right_logical(j, 5)
                row = lax.shift_right_logical(w_ix, 7)
                col = lax.bitwise_and(w_ix, 127)
                w = bits_v[row, pl.ds(col, 1)][0]
                bit = lax.bitwise_and(
                    lax.shift_right_logical(w, lax.bitwise_and(j, 31)), 1)
                off = lax.mul(bit, d)
                for q in range(d // 16):
                    compact_v[p, k, j, pl.ds(16 * q, 16)] = (
                        packed_v[p, k, j, pl.ds(off + 16 * q, 16)])
                return carry

            lax.fori_loop(0, CHUNK, body, 0)

        def fire_gathers(t, p):
            for k in range(K):
                gather(t, p, k).start()

        def drain_extract_write(t, p):
            for k in range(K):
                gather(t, p, k).wait()
                extract(t, p, k)
                write(t, p, k).start()

        def drain_writes(t, p):
            for k in range(K):
                write(t, p, k).wait()

        fire_gathers(0, 0)
        drain_extract_write(0, 0)
        fire_gathers(1, 1)

        def body(i, carry):
            for p in (1, 0):
                t = 2 * i + 1 + (1 - p)
                drain_extract_write(t, p)
                drain_writes(t - 1, 1 - p)
                fire_gathers(t + 1, 1 - p)
            return carry

        lax.fori_loop(0, (n_groups - 2) // 2, body, 0)

        t_last = n_groups - 1
        drain_extract_write(t_last, 1)
        drain_writes(t_last - 1, 0)
        drain_writes(t_last, 1)

    return lookup, nw, n_chunks, nbr


def kernel(x, embedding_weight):
    b, l = x.shape
    n_idx = b * l
    lookup, nw, n_chunks, nbr = _make_lookup(n_idx, D_MODEL)
    xi = x.astype(jnp.int32).reshape(nw, n_chunks, CHUNK)
    idxp = xi >> 1
    par = xi & 1
    shifts = jnp.arange(32, dtype=jnp.int32)
    bits = (par.reshape(nw, n_chunks * NBW, 32) << shifts).sum(
        axis=-1, dtype=jnp.int32)
    bits = jnp.pad(bits, ((0, 0), (0, nbr * 128 - n_chunks * NBW)))
    bits = bits.reshape(nw, nbr, 128)
    table2 = embedding_weight.reshape(embedding_weight.shape[0] // 2,
                                      2 * D_MODEL)
    out = lookup(idxp, bits, table2)
    return out.reshape(b, l, D_MODEL)
